# baseline (speedup 1.0000x reference)
import jax
import jax.numpy as jnp
from jax import lax
from jax.experimental import pallas as pl
from jax.experimental.pallas import tpu as pltpu
from jax.experimental.pallas import tpu_sc as plsc

_CHUNK = 2048


def kernel(batch, index):
    del index
    total, d = batch.shape
    n = total // _CHUNK

    info = plsc.get_sparse_core_info()
    nw = info.num_cores * info.num_subcores
    rows_per_w = total // nw
    halves = _CHUNK // rows_per_w

    mesh = plsc.VectorSubcoreMesh(core_axis_name="c", subcore_axis_name="s")

    def body(in_hbm, *args):
        outs = args[:n]
        sem = args[n]
        wid = lax.axis_index("s") * info.num_cores + lax.axis_index("c")
        local = (wid % halves) * rows_per_w
        for i in range(n):
            @pl.when(wid // halves == i)
            def _():
                cp = pltpu.make_async_copy(
                    in_hbm.at[pl.ds(i * _CHUNK + local, rows_per_w)],
                    outs[i].at[pl.ds(local, rows_per_w)],
                    sem,
                )
                cp.start()
                cp.wait()

    run = pl.kernel(
        body,
        out_type=tuple(
            jax.ShapeDtypeStruct((_CHUNK, d), batch.dtype) for _ in range(n)
        ),
        mesh=mesh,
        scratch_types=[pltpu.SemaphoreType.DMA],
    )
    return run(batch)


# speedup vs baseline: 1.0043x; 1.0043x over previous
import jax
import jax.numpy as jnp
from jax.experimental import pallas as pl
from jax.experimental.pallas import tpu as pltpu

_CHUNK = 2048


def kernel(batch, index):
    del index
    total, d = batch.shape
    n = total // _CHUNK

    def body(in_hbm, *args):
        outs = args[:n]
        sem = args[n]
        copies = []
        for i in range(n):
            cp = pltpu.make_async_copy(
                in_hbm.at[pl.ds(i * _CHUNK, _CHUNK)], outs[i], sem.at[i]
            )
            cp.start()
            copies.append(cp)
        for cp in copies:
            cp.wait()

    return pl.pallas_call(
        body,
        in_specs=[pl.BlockSpec(memory_space=pl.ANY)],
        out_specs=tuple(pl.BlockSpec(memory_space=pl.ANY) for _ in range(n)),
        out_shape=tuple(
            jax.ShapeDtypeStruct((_CHUNK, d), batch.dtype) for _ in range(n)
        ),
        scratch_shapes=[pltpu.SemaphoreType.DMA((n,))],
    )(batch)


# speedup vs baseline: 28.0020x; 27.8834x over previous
import jax
import jax.numpy as jnp
from jax.experimental import pallas as pl
from jax.experimental.pallas import tpu as pltpu

_CHUNK = 2048
_BLOCK = 256


def kernel(batch, index):
    del index
    total, d = batch.shape
    n = total // _CHUNK
    bpc = _CHUNK // _BLOCK
    steps = total // _BLOCK

    def body(in_ref, *out_refs):
        cid = pl.program_id(0) // bpc
        for i in range(n):
            @pl.when(cid == i)
            def _():
                out_refs[i][...] = in_ref[...]

    def out_spec(i):
        return pl.BlockSpec(
            (_BLOCK, d),
            lambda k, i=i: (jnp.clip(k - i * bpc, 0, bpc - 1), 0),
        )

    return pl.pallas_call(
        body,
        grid=(steps,),
        in_specs=[pl.BlockSpec((_BLOCK, d), lambda k: (k, 0))],
        out_specs=tuple(out_spec(i) for i in range(n)),
        out_shape=tuple(
            jax.ShapeDtypeStruct((_CHUNK, d), batch.dtype) for _ in range(n)
        ),
        compiler_params=pltpu.CompilerParams(
            dimension_semantics=("arbitrary",),
        ),
    )(batch)


# speedup vs baseline: 48.8581x; 1.7448x over previous
import jax
import jax.numpy as jnp
from jax.experimental import pallas as pl
from jax.experimental.pallas import tpu as pltpu

_CHUNK = 2048
_ROWS = 512
_NBUF = 8
_LAG = 4


def kernel(batch, index):
    del index
    total, d = batch.shape
    n = total // _CHUNK
    bpc = _CHUNK // _ROWS
    nblk = total // _ROWS

    def body(in_hbm, *args):
        outs = args[:n]
        buf, in_sem, out_sem = args[n], args[n + 1], args[n + 2]

        def in_copy(k):
            return pltpu.make_async_copy(
                in_hbm.at[pl.ds(k * _ROWS, _ROWS)],
                buf.at[k % _NBUF],
                in_sem.at[k % _NBUF],
            )

        def out_copy(k):
            return pltpu.make_async_copy(
                buf.at[k % _NBUF],
                outs[k // bpc].at[pl.ds((k % bpc) * _ROWS, _ROWS)],
                out_sem.at[k % _NBUF],
            )

        for k in range(_NBUF):
            in_copy(k).start()

        out_waited = [False] * nblk
        for k in range(nblk):
            in_copy(k).wait()
            out_copy(k).start()
            j = k - _LAG
            if j >= 0 and j + _NBUF < nblk:
                out_copy(j).wait()
                out_waited[j] = True
                in_copy(j + _NBUF).start()
        for k in range(nblk):
            if not out_waited[k]:
                out_copy(k).wait()

    return pl.pallas_call(
        body,
        in_specs=[pl.BlockSpec(memory_space=pl.ANY)],
        out_specs=tuple(pl.BlockSpec(memory_space=pl.ANY) for _ in range(n)),
        out_shape=tuple(
            jax.ShapeDtypeStruct((_CHUNK, d), batch.dtype) for _ in range(n)
        ),
        scratch_shapes=[
            pltpu.VMEM((_NBUF, _ROWS, d), batch.dtype),
            pltpu.SemaphoreType.DMA((_NBUF,)),
            pltpu.SemaphoreType.DMA((_NBUF,)),
        ],
    )(batch)
